# baseline (speedup 1.0000x reference)
import functools

import jax
import jax.numpy as jnp
from jax import lax
from jax.experimental import pallas as pl
from jax.experimental.pallas import tpu as pltpu
from jax.experimental.pallas import tpu_sc as plsc

NN = 10000
EE = 320000
DD = 128
NC = 2
NS = 16
NW = NC * NS
NPAD = 10240
RPT = NPAD // NS
EC = 64
EG = EC // 16
EPT = EE // NW
NCHUNK = 160
NGCHUNK = NCHUNK + 2

_MESH = plsc.VectorSubcoreMesh(core_axis_name="c", subcore_axis_name="s")
_SC_PARAMS = pltpu.CompilerParams(use_tc_tiling_on_sc=False)


def _unpack_slab(pk_v, dst_v, nchunk):
    def _body(i, carry):
        r = i // EG
        sl = pl.ds((i % EG) * 16, 16)
        v = pk_v[r, sl]
        dst_v[r, sl] = lax.shift_right_logical(v, 14)
        pk_v[r, sl] = v & 0x3FFF
        return carry

    lax.fori_loop(0, nchunk * EG, _body, 0)


@functools.partial(
    pl.kernel,
    out_type=jax.ShapeDtypeStruct((NC, NPAD), jnp.float32),
    mesh=_MESH,
    compiler_params=_SC_PARAMS,
    scratch_types=[
        pltpu.VMEM((NCHUNK, EC), jnp.int32),
        pltpu.VMEM((NCHUNK, EC), jnp.int32),
        pltpu.VMEM((NCHUNK, EC), jnp.float32),
        pltpu.VMEM((RPT,), jnp.float32),
        pltpu.VMEM_SHARED((NPAD,), jnp.float32),
    ],
)
def _deg_kernel(pk_hbm, w_hbm, deg_hbm, pk_v, dst_v, w_v, zbuf, acc):
    c = lax.axis_index("c")
    s = lax.axis_index("s")
    wid = c * NS + s
    pltpu.sync_copy(pk_hbm.at[wid, pl.ds(0, NCHUNK)], pk_v)
    pltpu.sync_copy(w_hbm.at[wid], w_v)
    _unpack_slab(pk_v, dst_v, NCHUNK)

    def _zero(i, carry):
        zbuf[pl.ds(i * 16, 16)] = jnp.zeros((16,), jnp.float32)
        return carry

    lax.fori_loop(0, RPT // 16, _zero, 0)
    pltpu.sync_copy(zbuf, acc.at[pl.ds(s * RPT, RPT)])
    plsc.subcore_barrier()

    def _chunk(j, carry):
        pltpu.sync_copy(w_v.at[j], acc.at[dst_v.at[j]], add=True)
        return carry

    lax.fori_loop(0, NCHUNK, _chunk, 0)
    plsc.subcore_barrier()
    pltpu.sync_copy(acc.at[pl.ds(s * RPT, RPT)],
                    deg_hbm.at[c, pl.ds(s * RPT, RPT)])


@functools.partial(
    pl.kernel,
    out_type=jax.ShapeDtypeStruct((NC, NPAD, DD), jnp.float32),
    mesh=_MESH,
    compiler_params=_SC_PARAMS,
    scratch_types=[
        pltpu.VMEM((NGCHUNK, EC), jnp.int32),
        pltpu.VMEM((4, EC), jnp.int32),
        pltpu.VMEM((4, EC), jnp.int32),
        pltpu.VMEM((4, EC), jnp.float32),
        pltpu.VMEM((4, EC, DD), jnp.float32),
        [pltpu.SemaphoreType.DMA] * 4,
        [pltpu.SemaphoreType.DMA] * 4,
        [pltpu.SemaphoreType.DMA] * 4,
        pltpu.VMEM_SHARED((NPAD, DD), jnp.float32),
    ],
)
def _spmm_kernel(pk_hbm, w_hbm, u_hbm, out_hbm,
                 pk_v, sidx, didx, w_v, rows_v, gsems, ssems, wsems, acc):
    c = lax.axis_index("c")
    s = lax.axis_index("s")
    wid = c * NS + s

    def _ldpk(i, carry):
        pltpu.sync_copy(pk_hbm.at[wid, pl.ds(i * 2, 2)],
                        pk_v.at[pl.ds(i * 2, 2)])
        return carry

    lax.fori_loop(0, NGCHUNK // 2, _ldpk, 0)

    def _zero(i, carry):
        for g in range(DD // 16):
            rows_v[0, i, pl.ds(g * 16, 16)] = jnp.zeros((16,), jnp.float32)
        return carry

    lax.fori_loop(0, EC, _zero, 0)
    for k in range(RPT // EC):
        pltpu.sync_copy(rows_v.at[0], acc.at[pl.ds(s * RPT + k * EC, EC)])
    plsc.subcore_barrier()

    def _unpack_chunk(jj, b):
        def _g(g, carry):
            sl = pl.ds(g * 16, 16)
            v = pk_v[jj, sl]
            didx[b, sl] = lax.shift_right_logical(v, 14)
            sidx[b, sl] = v & 0x3FFF
            return carry

        lax.fori_loop(0, EG, _g, 0, unroll=True)

    def _fire_gather(jj, b):
        pltpu.async_copy(u_hbm.at[sidx.at[b]], rows_v.at[b], gsems[b])

    def _fire_wload(jj, b):
        pltpu.async_copy(w_hbm.at[wid, jj], w_v.at[b], wsems[b])

    def _wait_gather(jj, b):
        pltpu.make_async_copy(u_hbm.at[sidx.at[b]], rows_v.at[b],
                              gsems[b]).wait()

    def _wait_wload(jj, b):
        pltpu.make_async_copy(w_hbm.at[wid, jj], w_v.at[b], wsems[b]).wait()

    def _fire_scatter(jj, b):
        pltpu.async_copy(rows_v.at[b], acc.at[didx.at[b]], ssems[b],
                         add=True)

    def _wait_scatter(jj, b):
        pltpu.make_async_copy(rows_v.at[b], acc.at[didx.at[b]],
                              ssems[b]).wait()

    def _scale(jj, b):
        NG = DD // 16

        def _grp(g16, carry):
            w16 = w_v[b, pl.ds(g16 * 16, 16)]
            for e4 in range(4):
                base = g16 * 16 + e4 * 4
                wls = [w16[e4 * 4 + i] for i in range(4)]
                vals = [rows_v[b, base + i, pl.ds(g * 16, 16)]
                        for i in range(4) for g in range(NG)]
                prods = [v * wls[i // NG] for i, v in enumerate(vals)]
                for i in range(4):
                    for g in range(NG):
                        rows_v[b, base + i, pl.ds(g * 16, 16)] = (
                            prods[i * NG + g])
            return carry

        lax.fori_loop(0, EC // 16, _grp, 0, unroll=2)

    for jj in range(2):
        _unpack_chunk(jj, jj)
        _fire_gather(jj, jj)
        _fire_wload(jj, jj)

    def _slot(jj, b):
        b2 = (b + 2) % 4

        @pl.when(jnp.logical_and(jj >= 2, jj < NCHUNK + 2))
        def _():
            _wait_scatter(jj - 2, b2)

        @pl.when(jj < NCHUNK)
        def _():
            _unpack_chunk(jj + 2, b2)
            _fire_gather(jj + 2, b2)

        @pl.when(jj < NCHUNK - 2)
        def _():
            _fire_wload(jj + 2, b2)

        @pl.when(jj < NCHUNK)
        def _():
            _wait_gather(jj, b)
            _wait_wload(jj, b)
            _scale(jj, b)
            _fire_scatter(jj, b)

        @pl.when(jnp.logical_and(jj >= NCHUNK, jj < NCHUNK + 2))
        def _():
            _wait_gather(jj, b)

    def _outer(i, carry):
        for bo in range(4):
            _slot(i * 4 + bo, bo)
        return carry

    lax.fori_loop(0, (NCHUNK + 4) // 4, _outer, 0)
    plsc.subcore_barrier()

    def _st(i, carry):
        pltpu.sync_copy(acc.at[pl.ds(s * RPT + i * 16, 16)],
                        out_hbm.at[c, pl.ds(s * RPT + i * 16, 16)])
        return carry

    lax.fori_loop(0, RPT // 16, _st, 0)


_RB = 1000
_GRID = NN // _RB


def _pre_body(x_ref, w_ref, degp_ref, u_ref, dis_ref):
    deg = degp_ref[0] + degp_ref[1] + 1.0
    dis = lax.rsqrt(deg)
    u_ref[...] = jnp.dot(x_ref[...], w_ref[...],
                         preferred_element_type=jnp.float32) * dis
    dis_ref[...] = dis


def _mid_body(sp_ref, u_ref, dis_ref, b_ref, w_ref, o_ref):
    t = sp_ref[0] + sp_ref[1] + u_ref[...]
    q = jnp.maximum(t * dis_ref[...] + b_ref[...], 0.0)
    o_ref[...] = jnp.dot(q, w_ref[...],
                         preferred_element_type=jnp.float32) * dis_ref[...]


def _post_body(sp_ref, u_ref, dis_ref, b_ref, out_ref):
    t = sp_ref[0] + sp_ref[1] + u_ref[...]
    out_ref[...] = t * dis_ref[...] + b_ref[...]


def _tc_pre(x, W1, degp2d):
    return pl.pallas_call(
        _pre_body,
        grid=(_GRID,),
        in_specs=[
            pl.BlockSpec((_RB, DD), lambda i: (i, 0)),
            pl.BlockSpec((DD, DD), lambda i: (0, 0)),
            pl.BlockSpec((2, _RB, 1), lambda i: (0, i, 0)),
        ],
        out_specs=[
            pl.BlockSpec((_RB, DD), lambda i: (i, 0)),
            pl.BlockSpec((_RB, 1), lambda i: (i, 0)),
        ],
        out_shape=[
            jax.ShapeDtypeStruct((NN, DD), jnp.float32),
            jax.ShapeDtypeStruct((NN, 1), jnp.float32),
        ],
    )(x, W1, degp2d)


def _tc_mid(sp, u, dis, b, W):
    return pl.pallas_call(
        _mid_body,
        grid=(_GRID,),
        in_specs=[
            pl.BlockSpec((2, _RB, DD), lambda i: (0, i, 0)),
            pl.BlockSpec((_RB, DD), lambda i: (i, 0)),
            pl.BlockSpec((_RB, 1), lambda i: (i, 0)),
            pl.BlockSpec((1, DD), lambda i: (0, 0)),
            pl.BlockSpec((DD, DD), lambda i: (0, 0)),
        ],
        out_specs=pl.BlockSpec((_RB, DD), lambda i: (i, 0)),
        out_shape=jax.ShapeDtypeStruct((NN, DD), jnp.float32),
    )(sp, u, dis, b, W)


def _tc_post(sp, u, dis, b):
    return pl.pallas_call(
        _post_body,
        grid=(_GRID,),
        in_specs=[
            pl.BlockSpec((2, _RB, DD), lambda i: (0, i, 0)),
            pl.BlockSpec((_RB, DD), lambda i: (i, 0)),
            pl.BlockSpec((_RB, 1), lambda i: (i, 0)),
            pl.BlockSpec((1, DD), lambda i: (0, 0)),
        ],
        out_specs=pl.BlockSpec((_RB, DD), lambda i: (i, 0)),
        out_shape=jax.ShapeDtypeStruct((NN, DD), jnp.float32),
    )(sp, u, dis, b)


def _build_edge_slabs(src, dst, w):
    padlen = NCHUNK * EC - EPT
    srcs = src.reshape(NW, EPT)
    dsts = dst.reshape(NW, EPT)
    ws = w.reshape(NW, EPT)
    pad_src = ((jnp.arange(padlen, dtype=jnp.int32) * 41) % NN)[None, :]
    pad_dst = (NN + jnp.arange(padlen, dtype=jnp.int32) % (NPAD - NN))[None, :]
    pad_w = jnp.zeros((1, padlen), jnp.float32)
    srcs = jnp.concatenate(
        [srcs, jnp.broadcast_to(pad_src, (NW, padlen))], axis=1)
    dsts = jnp.concatenate(
        [dsts, jnp.broadcast_to(pad_dst, (NW, padlen))], axis=1)
    ws = jnp.concatenate([ws, jnp.broadcast_to(pad_w, (NW, padlen))], axis=1)
    d_src = jnp.broadcast_to(
        ((jnp.arange(2 * EC, dtype=jnp.int32) * 79) % NN)[None, :],
        (NW, 2 * EC))
    d_dst = jnp.broadcast_to(
        (NN + jnp.arange(2 * EC, dtype=jnp.int32) % (NPAD - NN))[None, :],
        (NW, 2 * EC))
    srcs_g = jnp.concatenate([srcs, d_src], axis=1)
    dsts_g = jnp.concatenate([dsts, d_dst], axis=1)
    packed = (srcs_g | (dsts_g << 14)).reshape(NW, NGCHUNK, EC)
    w_slab = ws.reshape(NW, NCHUNK, EC)
    return packed, w_slab


def kernel(x, edge_index, edge_attr, W1, b1, W2, b2, W3, b3, W4, b4):
    src = edge_index[0]
    dst = edge_index[1]
    pk_slab, w_slab = _build_edge_slabs(src, dst, edge_attr)

    degp = _deg_kernel(pk_slab, w_slab)
    degp2d = degp[:, :NN, None]

    u1, dis = _tc_pre(x, W1, degp2d)
    s1 = _spmm_kernel(pk_slab, w_slab, u1)
    u2 = _tc_mid(s1, u1, dis, b1.reshape(1, DD), W2)
    s2 = _spmm_kernel(pk_slab, w_slab, u2)
    u3 = _tc_mid(s2, u2, dis, b2.reshape(1, DD), W3)
    s3 = _spmm_kernel(pk_slab, w_slab, u3)
    u4 = _tc_mid(s3, u3, dis, b3.reshape(1, DD), W4)
    s4 = _spmm_kernel(pk_slab, w_slab, u4)
    return _tc_post(s4, u4, dis, b4.reshape(1, DD))
